# Initial kernel scaffold; baseline (speedup 1.0000x reference)
import jax
import jax.numpy as jnp
from jax.experimental import pallas as pl


def kernel(x, edge_index, W1, b1, g1, be1, W2, b2, g2, be2, W3, b3):
    raise NotImplementedError("write your pallas kernel here")


# speedup vs baseline: 6.9839x; 6.9839x over previous
import functools

import jax
import jax.numpy as jnp
from jax import lax
from jax.experimental import pallas as pl
from jax.experimental.pallas import tpu as pltpu
from jax.experimental.pallas import tpu_sc as plsc

N = 10000
E = 320000
D = 128
EPS = 1e-5

NTILES = 32
CHUNK = 128
G = 8
KCH = 80
NGRP = KCH // G
EPT = KCH * CHUNK
EPAD = NTILES * EPT
NPAD = 10112
RPT = NPAD // 16
RBLK = 1000
GRID = N // RBLK

_mesh = plsc.VectorSubcoreMesh(core_axis_name="c", subcore_axis_name="s")


@functools.partial(
    pl.kernel,
    out_type=jax.ShapeDtypeStruct((2, NPAD, D), jnp.float32),
    mesh=_mesh,
    scratch_types=[
        pltpu.VMEM((G, CHUNK), jnp.int32),
        pltpu.VMEM((CHUNK, D), jnp.float32),
        pltpu.VMEM_SHARED((NPAD, D), jnp.float32),
    ],
)
def _deg_kernel(dst_hbm, ones_hbm, zrows_hbm, out_hbm, dst_g, ones_v, acc):
    cid = lax.axis_index("c")
    sid = lax.axis_index("s")
    wid = sid * 2 + cid
    sl = pl.ds(sid * RPT, RPT)
    pltpu.sync_copy(ones_hbm, ones_v)
    pltpu.sync_copy(zrows_hbm.at[sl], acc.at[sl])
    plsc.subcore_barrier()

    def group(g, carry):
        pltpu.sync_copy(dst_hbm.at[wid, pl.ds(g * G, G)], dst_g)

        def body(jj, c2):
            pltpu.sync_copy(ones_v, acc.at[dst_g.at[jj]], add=True)
            return c2

        return lax.fori_loop(0, G, body, carry)

    lax.fori_loop(0, NGRP, group, 0)
    plsc.subcore_barrier()
    pltpu.sync_copy(acc.at[sl], out_hbm.at[cid, sl])


@functools.partial(
    pl.kernel,
    out_type=jax.ShapeDtypeStruct((2, NPAD, D), jnp.float32),
    mesh=_mesh,
    scratch_types=[
        pltpu.VMEM((G, CHUNK), jnp.int32),
        pltpu.VMEM((G, CHUNK), jnp.int32),
        pltpu.VMEM((CHUNK, D), jnp.float32),
        pltpu.VMEM_SHARED((NPAD, D), jnp.float32),
        pltpu.SemaphoreType.DMA,
    ],
)
def _agg_kernel(h_hbm, src_hbm, dst_hbm, zrows_hbm, out_hbm,
                src_g, dst_g, rows, acc, sem):
    cid = lax.axis_index("c")
    sid = lax.axis_index("s")
    wid = sid * 2 + cid
    sl = pl.ds(sid * RPT, RPT)
    pltpu.sync_copy(zrows_hbm.at[sl], acc.at[sl])
    plsc.subcore_barrier()

    def group(g, carry):
        pltpu.sync_copy(src_hbm.at[wid, pl.ds(g * G, G)], src_g)
        pltpu.sync_copy(dst_hbm.at[wid, pl.ds(g * G, G)], dst_g)

        def body(jj, c2):
            pltpu.async_copy(h_hbm.at[src_g.at[jj]], rows, sem).wait()
            pltpu.sync_copy(rows, acc.at[dst_g.at[jj]], add=True)
            return c2

        return lax.fori_loop(0, G, body, carry)

    lax.fori_loop(0, NGRP, group, 0)
    plsc.subcore_barrier()
    pltpu.sync_copy(acc.at[sl], out_hbm.at[cid, sl])


def _tc1_body(parts_ref, x_ref, w_ref, dis_ref, h_ref):
    p0 = parts_ref[0]
    p1 = parts_ref[1]
    deg = (p0 + p1)[:, 0:1] + 1.0
    dis = jnp.broadcast_to(lax.rsqrt(deg), (RBLK, D))
    h = jnp.dot(x_ref[...], w_ref[...], preferred_element_type=jnp.float32,
                precision=lax.Precision.HIGHEST)
    dis_ref[...] = dis
    h_ref[...] = dis * h


def _tc_mid_body(agg_ref, hp_ref, dis_ref, w_ref, st_ref, res_ref,
                 x_ref, h_ref, *, residual):
    a = agg_ref[0] + agg_ref[1] + hp_ref[...]
    dis = dis_ref[...]
    y = a * dis * st_ref[0] + st_ref[1]
    x = jnp.maximum(y, 0.0)
    if residual:
        x = x + res_ref[...]
    h = jnp.dot(x, w_ref[...], preferred_element_type=jnp.float32,
                precision=lax.Precision.HIGHEST)
    x_ref[...] = x
    h_ref[...] = dis * h


def _tc4_body(agg_ref, hp_ref, dis_ref, b_ref, out_ref):
    a = agg_ref[0] + agg_ref[1] + hp_ref[...]
    out_ref[...] = a * dis_ref[...] + b_ref[0]


def _row_spec(shape3=False):
    if shape3:
        return pl.BlockSpec((2, RBLK, D), lambda i: (0, i, 0))
    return pl.BlockSpec((RBLK, D), lambda i: (i, 0))


_full_w = pl.BlockSpec((D, D), lambda i: (0, 0))
_vec_spec = pl.BlockSpec((1, D), lambda i: (0, 0))


def kernel(x, edge_index, W1, b1, g1, be1, W2, b2, g2, be2, W3, b3):
    ei = edge_index.astype(jnp.int32)
    pad = EPAD - E
    src = jnp.concatenate([ei[0], jnp.zeros((pad,), jnp.int32)])
    dst = jnp.concatenate([ei[1], jnp.full((pad,), N, jnp.int32)])
    src = src.reshape(NTILES, KCH, CHUNK)
    dst = dst.reshape(NTILES, KCH, CHUNK)

    ones_rows = jnp.ones((CHUNK, D), jnp.float32)
    zrows = jnp.zeros((NPAD, D), jnp.float32)

    sc = 1.0 / jnp.sqrt(1.0 + EPS)
    st1 = jnp.stack([g1 * sc, b1 * g1 * sc + be1]).reshape(2, 1, D)
    st2 = jnp.stack([g2 * sc, b2 * g2 * sc + be2]).reshape(2, 1, D)

    deg_parts = _deg_kernel(dst, ones_rows, zrows)

    dis, h1 = pl.pallas_call(
        _tc1_body,
        grid=(GRID,),
        in_specs=[_row_spec(True),
                  _row_spec(), _full_w],
        out_specs=[_row_spec(), _row_spec()],
        out_shape=[jax.ShapeDtypeStruct((N, D), jnp.float32),
                   jax.ShapeDtypeStruct((N, D), jnp.float32)],
    )(deg_parts, x, W1)

    agg1 = _agg_kernel(h1, src, dst, zrows)

    st_spec = pl.BlockSpec((2, 1, D), lambda i: (0, 0, 0))
    x1, h2 = pl.pallas_call(
        functools.partial(_tc_mid_body, residual=False),
        grid=(GRID,),
        in_specs=[_row_spec(True), _row_spec(), _row_spec(), _full_w,
                  st_spec, _row_spec()],
        out_specs=[_row_spec(), _row_spec()],
        out_shape=[jax.ShapeDtypeStruct((N, D), jnp.float32),
                   jax.ShapeDtypeStruct((N, D), jnp.float32)],
    )(agg1, h1, dis, W2, st1, h1)

    agg2 = _agg_kernel(h2, src, dst, zrows)

    x2, h3 = pl.pallas_call(
        functools.partial(_tc_mid_body, residual=True),
        grid=(GRID,),
        in_specs=[_row_spec(True), _row_spec(), _row_spec(), _full_w,
                  st_spec, _row_spec()],
        out_specs=[_row_spec(), _row_spec()],
        out_shape=[jax.ShapeDtypeStruct((N, D), jnp.float32),
                   jax.ShapeDtypeStruct((N, D), jnp.float32)],
    )(agg2, h2, dis, W3, st2, x1)

    agg3 = _agg_kernel(h3, src, dst, zrows)

    out = pl.pallas_call(
        _tc4_body,
        grid=(GRID,),
        in_specs=[_row_spec(True), _row_spec(), _row_spec(), _vec_spec],
        out_specs=_row_spec(),
        out_shape=jax.ShapeDtypeStruct((N, D), jnp.float32),
    )(agg3, h3, dis, b3.reshape(1, D))

    return out


# speedup vs baseline: 7.6067x; 1.0892x over previous
import functools

import jax
import jax.numpy as jnp
from jax import lax
from jax.experimental import pallas as pl
from jax.experimental.pallas import tpu as pltpu
from jax.experimental.pallas import tpu_sc as plsc

N = 10000
E = 320000
D = 128
EPS = 1e-5

NTILES = 32
CHUNK = 128
G = 8
KCH = 80
NGRP = KCH // G
EPT = KCH * CHUNK
EPAD = NTILES * EPT
NPAD = 10112
RPT = NPAD // 16
RBLK = 1000
GRID = N // RBLK

_mesh = plsc.VectorSubcoreMesh(core_axis_name="c", subcore_axis_name="s")


@functools.partial(
    pl.kernel,
    out_type=jax.ShapeDtypeStruct((2, NPAD, D), jnp.float32),
    mesh=_mesh,
    scratch_types=[
        pltpu.VMEM((G, CHUNK), jnp.int32),
        pltpu.VMEM((CHUNK, D), jnp.float32),
        pltpu.VMEM_SHARED((NPAD, D), jnp.float32),
    ],
)
def _deg_kernel(dst_hbm, ones_hbm, zrows_hbm, out_hbm, dst_g, ones_v, acc):
    cid = lax.axis_index("c")
    sid = lax.axis_index("s")
    wid = sid * 2 + cid
    sl = pl.ds(sid * RPT, RPT)
    pltpu.sync_copy(ones_hbm, ones_v)
    pltpu.sync_copy(zrows_hbm.at[sl], acc.at[sl])
    plsc.subcore_barrier()

    def group(g, carry):
        pltpu.sync_copy(dst_hbm.at[wid, pl.ds(g * G, G)], dst_g)

        def body(jj, c2):
            pltpu.sync_copy(ones_v, acc.at[dst_g.at[jj]], add=True)
            return c2

        return lax.fori_loop(0, G, body, carry)

    lax.fori_loop(0, NGRP, group, 0)
    plsc.subcore_barrier()
    pltpu.sync_copy(acc.at[sl], out_hbm.at[cid, sl])


@functools.partial(
    pl.kernel,
    out_type=jax.ShapeDtypeStruct((2, NPAD, D), jnp.float32),
    mesh=_mesh,
    scratch_types=[
        pltpu.VMEM((2, G, CHUNK), jnp.int32),
        pltpu.VMEM((2, G, CHUNK), jnp.int32),
        pltpu.VMEM((CHUNK, D), jnp.float32),
        pltpu.VMEM((CHUNK, D), jnp.float32),
        pltpu.VMEM_SHARED((NPAD, D), jnp.float32),
        pltpu.SemaphoreType.DMA,
        pltpu.SemaphoreType.DMA,
        pltpu.SemaphoreType.DMA,
        pltpu.SemaphoreType.DMA,
    ],
)
def _agg_kernel(h_hbm, src_hbm, dst_hbm, zrows_hbm, out_hbm,
                src_i, dst_i, rows0, rows1, acc, sg0, sg1, ss0, ss1):
    cid = lax.axis_index("c")
    sid = lax.axis_index("s")
    wid = sid * 2 + cid
    sl = pl.ds(sid * RPT, RPT)
    rows = (rows0, rows1)
    sgs = (sg0, sg1)
    sss = (ss0, ss1)

    pltpu.sync_copy(zrows_hbm.at[sl], acc.at[sl])
    pltpu.sync_copy(src_hbm.at[wid, pl.ds(0, G)], src_i.at[0])
    pltpu.sync_copy(dst_hbm.at[wid, pl.ds(0, G)], dst_i.at[0])
    plsc.subcore_barrier()
    pltpu.async_copy(h_hbm.at[src_i.at[0, 0]], rows0, sg0)

    def grp(g, carry):
        g2 = lax.rem(g, 2)
        ng2 = lax.rem(g + 1, 2)

        @pl.when(g + 1 < NGRP)
        def _():
            pltpu.sync_copy(src_hbm.at[wid, pl.ds((g + 1) * G, G)],
                            src_i.at[ng2])
            pltpu.sync_copy(dst_hbm.at[wid, pl.ds((g + 1) * G, G)],
                            dst_i.at[ng2])

        for jj in range(G):
            b = jj % 2
            nb = (jj + 1) % 2
            pltpu.make_async_copy(h_hbm.at[src_i.at[g2, jj]],
                                  rows[b], sgs[b]).wait()

            def wait_prev():
                pltpu.make_async_copy(
                    rows[nb], acc.at[dst_i.at[g2, jj]], sss[nb]).wait()

            if jj == 0:
                pl.when(g > 0)(wait_prev)
            else:
                wait_prev()

            if jj + 1 < G:
                pltpu.async_copy(h_hbm.at[src_i.at[g2, jj + 1]],
                                 rows[nb], sgs[nb])
            else:
                @pl.when(g + 1 < NGRP)
                def _():
                    pltpu.async_copy(h_hbm.at[src_i.at[ng2, 0]],
                                     rows[nb], sgs[nb])

            pltpu.async_copy(rows[b], acc.at[dst_i.at[g2, jj]],
                             sss[b], add=True)
        return carry

    lax.fori_loop(0, NGRP, grp, 0)
    pltpu.make_async_copy(rows1, acc.at[dst_i.at[0, 0]], ss1).wait()
    plsc.subcore_barrier()
    pltpu.sync_copy(acc.at[sl], out_hbm.at[cid, sl])


def _tc1_body(parts_ref, x_ref, w_ref, dis_ref, h_ref):
    p0 = parts_ref[0]
    p1 = parts_ref[1]
    deg = (p0 + p1)[:, 0:1] + 1.0
    dis = jnp.broadcast_to(lax.rsqrt(deg), (RBLK, D))
    h = jnp.dot(x_ref[...], w_ref[...], preferred_element_type=jnp.float32,
                precision=lax.Precision.HIGHEST)
    dis_ref[...] = dis
    h_ref[...] = dis * h


def _tc_mid_body(agg_ref, hp_ref, dis_ref, w_ref, st_ref, res_ref,
                 x_ref, h_ref, *, residual):
    a = agg_ref[0] + agg_ref[1] + hp_ref[...]
    dis = dis_ref[...]
    y = a * dis * st_ref[0] + st_ref[1]
    x = jnp.maximum(y, 0.0)
    if residual:
        x = x + res_ref[...]
    h = jnp.dot(x, w_ref[...], preferred_element_type=jnp.float32,
                precision=lax.Precision.HIGHEST)
    x_ref[...] = x
    h_ref[...] = dis * h


def _tc4_body(agg_ref, hp_ref, dis_ref, b_ref, out_ref):
    a = agg_ref[0] + agg_ref[1] + hp_ref[...]
    out_ref[...] = a * dis_ref[...] + b_ref[0]


def _row_spec(shape3=False):
    if shape3:
        return pl.BlockSpec((2, RBLK, D), lambda i: (0, i, 0))
    return pl.BlockSpec((RBLK, D), lambda i: (i, 0))


_full_w = pl.BlockSpec((D, D), lambda i: (0, 0))
_vec_spec = pl.BlockSpec((1, D), lambda i: (0, 0))


def kernel(x, edge_index, W1, b1, g1, be1, W2, b2, g2, be2, W3, b3):
    ei = edge_index.astype(jnp.int32)
    pad = EPAD - E
    src = jnp.concatenate([ei[0], jnp.zeros((pad,), jnp.int32)])
    dst = jnp.concatenate([ei[1], jnp.full((pad,), N, jnp.int32)])
    src = src.reshape(NTILES, KCH, CHUNK)
    dst = dst.reshape(NTILES, KCH, CHUNK)

    ones_rows = jnp.ones((CHUNK, D), jnp.float32)
    zrows = jnp.zeros((NPAD, D), jnp.float32)

    sc = 1.0 / jnp.sqrt(1.0 + EPS)
    st1 = jnp.stack([g1 * sc, b1 * g1 * sc + be1]).reshape(2, 1, D)
    st2 = jnp.stack([g2 * sc, b2 * g2 * sc + be2]).reshape(2, 1, D)

    deg_parts = _deg_kernel(dst, ones_rows, zrows)

    dis, h1 = pl.pallas_call(
        _tc1_body,
        grid=(GRID,),
        in_specs=[_row_spec(True),
                  _row_spec(), _full_w],
        out_specs=[_row_spec(), _row_spec()],
        out_shape=[jax.ShapeDtypeStruct((N, D), jnp.float32),
                   jax.ShapeDtypeStruct((N, D), jnp.float32)],
    )(deg_parts, x, W1)

    agg1 = _agg_kernel(h1, src, dst, zrows)

    st_spec = pl.BlockSpec((2, 1, D), lambda i: (0, 0, 0))
    x1, h2 = pl.pallas_call(
        functools.partial(_tc_mid_body, residual=False),
        grid=(GRID,),
        in_specs=[_row_spec(True), _row_spec(), _row_spec(), _full_w,
                  st_spec, _row_spec()],
        out_specs=[_row_spec(), _row_spec()],
        out_shape=[jax.ShapeDtypeStruct((N, D), jnp.float32),
                   jax.ShapeDtypeStruct((N, D), jnp.float32)],
    )(agg1, h1, dis, W2, st1, h1)

    agg2 = _agg_kernel(h2, src, dst, zrows)

    x2, h3 = pl.pallas_call(
        functools.partial(_tc_mid_body, residual=True),
        grid=(GRID,),
        in_specs=[_row_spec(True), _row_spec(), _row_spec(), _full_w,
                  st_spec, _row_spec()],
        out_specs=[_row_spec(), _row_spec()],
        out_shape=[jax.ShapeDtypeStruct((N, D), jnp.float32),
                   jax.ShapeDtypeStruct((N, D), jnp.float32)],
    )(agg2, h2, dis, W3, st2, x1)

    agg3 = _agg_kernel(h3, src, dst, zrows)

    out = pl.pallas_call(
        _tc4_body,
        grid=(GRID,),
        in_specs=[_row_spec(True), _row_spec(), _row_spec(), _vec_spec],
        out_specs=_row_spec(),
        out_shape=jax.ShapeDtypeStruct((N, D), jnp.float32),
    )(agg3, h3, dis, b3.reshape(1, D))

    return out


# speedup vs baseline: 9.3157x; 1.2247x over previous
import functools

import jax
import jax.numpy as jnp
from jax import lax
from jax.experimental import pallas as pl
from jax.experimental.pallas import tpu as pltpu
from jax.experimental.pallas import tpu_sc as plsc

N = 10000
E = 320000
D = 128
EPS = 1e-5

NTILES = 32
CHUNK = 128
G = 8
NGT = 20
NCHT = NGT * G
EPAD = 16 * NCHT * CHUNK
NG0 = 15
NG1 = NGT - NG0
NPAD = 10112
RPT = NPAD // 16
RBLK = 1000
GRID = N // RBLK

_mesh = plsc.VectorSubcoreMesh(core_axis_name="c", subcore_axis_name="s")


@functools.partial(
    pl.kernel,
    out_type=jax.ShapeDtypeStruct((2, NPAD, D), jnp.float32),
    mesh=_mesh,
    scratch_types=[
        pltpu.VMEM((G, CHUNK), jnp.int32),
        pltpu.VMEM((CHUNK, D), jnp.float32),
        pltpu.VMEM_SHARED((NPAD, D), jnp.float32),
    ],
)
def _deg_kernel(dst_hbm, ones_hbm, zrows_hbm, out_hbm, dst_g, ones_v, acc):
    cid = lax.axis_index("c")
    sid = lax.axis_index("s")
    gbase = cid * (NGT // 2)
    sl = pl.ds(sid * RPT, RPT)
    pltpu.sync_copy(ones_hbm, ones_v)
    pltpu.sync_copy(zrows_hbm.at[sl], acc.at[sl])
    plsc.subcore_barrier()

    def group(g, carry):
        pltpu.sync_copy(dst_hbm.at[sid, pl.ds((gbase + g) * G, G)], dst_g)

        def body(jj, c2):
            pltpu.sync_copy(ones_v, acc.at[dst_g.at[jj]], add=True)
            return c2

        return lax.fori_loop(0, G, body, carry)

    lax.fori_loop(0, NGT // 2, group, 0)
    plsc.subcore_barrier()
    pltpu.sync_copy(acc.at[sl], out_hbm.at[cid, sl])


@functools.partial(
    pl.kernel,
    out_type=jax.ShapeDtypeStruct((2, NPAD, D), jnp.float32),
    mesh=_mesh,
    scratch_types=[
        pltpu.VMEM((2, G, CHUNK), jnp.int32),
        pltpu.VMEM((2, G, CHUNK), jnp.int32),
        pltpu.VMEM((CHUNK, D), jnp.float32),
        pltpu.VMEM((CHUNK, D), jnp.float32),
        pltpu.VMEM_SHARED((NPAD, D), jnp.float32),
        pltpu.SemaphoreType.DMA,
        pltpu.SemaphoreType.DMA,
        pltpu.SemaphoreType.DMA,
        pltpu.SemaphoreType.DMA,
    ],
)
def _agg_kernel(h_hbm, src_hbm, dst_hbm, zrows_hbm, out_hbm,
                src_i, dst_i, rows0, rows1, acc, sg0, sg1, ss0, ss1):
    cid = lax.axis_index("c")
    sid = lax.axis_index("s")
    gbase = jnp.where(cid == 0, 0, NG0)
    ng = jnp.where(cid == 0, NG0, NG1)
    sl = pl.ds(sid * RPT, RPT)
    rows = (rows0, rows1)
    sgs = (sg0, sg1)
    sss = (ss0, ss1)

    pltpu.sync_copy(zrows_hbm.at[sl], acc.at[sl])
    pltpu.sync_copy(src_hbm.at[sid, pl.ds(gbase * G, G)], src_i.at[0])
    pltpu.sync_copy(dst_hbm.at[sid, pl.ds(gbase * G, G)], dst_i.at[0])
    plsc.subcore_barrier()
    pltpu.async_copy(h_hbm.at[src_i.at[0, 0]], rows0, sg0)

    def grp(g, carry):
        g2 = lax.rem(g, 2)
        ng2 = lax.rem(g + 1, 2)

        @pl.when(g + 1 < ng)
        def _():
            pltpu.sync_copy(src_hbm.at[sid, pl.ds((gbase + g + 1) * G, G)],
                            src_i.at[ng2])
            pltpu.sync_copy(dst_hbm.at[sid, pl.ds((gbase + g + 1) * G, G)],
                            dst_i.at[ng2])

        for jj in range(G):
            b = jj % 2
            nb = (jj + 1) % 2
            pltpu.make_async_copy(h_hbm.at[src_i.at[g2, jj]],
                                  rows[b], sgs[b]).wait()

            def wait_prev():
                pltpu.make_async_copy(
                    rows[nb], acc.at[dst_i.at[g2, jj]], sss[nb]).wait()

            if jj == 0:
                pl.when(g > 0)(wait_prev)
            else:
                wait_prev()

            if jj + 1 < G:
                pltpu.async_copy(h_hbm.at[src_i.at[g2, jj + 1]],
                                 rows[nb], sgs[nb])
            else:
                @pl.when(g + 1 < ng)
                def _():
                    pltpu.async_copy(h_hbm.at[src_i.at[ng2, 0]],
                                     rows[nb], sgs[nb])

            pltpu.async_copy(rows[b], acc.at[dst_i.at[g2, jj]],
                             sss[b], add=True)
        return carry

    lax.fori_loop(0, ng, grp, 0)
    pltpu.make_async_copy(rows1, acc.at[dst_i.at[0, 0]], ss1).wait()
    plsc.subcore_barrier()
    pltpu.sync_copy(acc.at[sl], out_hbm.at[cid, sl])


def _tc1_body(parts_ref, x_ref, w_ref, dis_ref, h_ref):
    p0 = parts_ref[0]
    p1 = parts_ref[1]
    deg = (p0 + p1)[:, 0:1] + 1.0
    dis = jnp.broadcast_to(lax.rsqrt(deg), (RBLK, D))
    h = jnp.dot(x_ref[...], w_ref[...], preferred_element_type=jnp.float32,
                precision=lax.Precision.HIGHEST)
    dis_ref[...] = dis
    h_ref[...] = dis * h


def _tc_mid_body(agg_ref, hp_ref, dis_ref, w_ref, st_ref, res_ref,
                 x_ref, h_ref, *, residual):
    a = agg_ref[0] + agg_ref[1] + hp_ref[...]
    dis = dis_ref[...]
    y = a * dis * st_ref[0] + st_ref[1]
    x = jnp.maximum(y, 0.0)
    if residual:
        x = x + res_ref[...]
    h = jnp.dot(x, w_ref[...], preferred_element_type=jnp.float32,
                precision=lax.Precision.HIGHEST)
    x_ref[...] = x
    h_ref[...] = dis * h


def _tc4_body(agg_ref, hp_ref, dis_ref, b_ref, out_ref):
    a = agg_ref[0] + agg_ref[1] + hp_ref[...]
    out_ref[...] = a * dis_ref[...] + b_ref[0]


def _row_spec(shape3=False):
    if shape3:
        return pl.BlockSpec((2, RBLK, D), lambda i: (0, i, 0))
    return pl.BlockSpec((RBLK, D), lambda i: (i, 0))


_full_w = pl.BlockSpec((D, D), lambda i: (0, 0))
_vec_spec = pl.BlockSpec((1, D), lambda i: (0, 0))


def kernel(x, edge_index, W1, b1, g1, be1, W2, b2, g2, be2, W3, b3):
    ei = edge_index.astype(jnp.int32)
    pad = EPAD - E
    src = jnp.concatenate([ei[0], jnp.zeros((pad,), jnp.int32)])
    dst = jnp.concatenate([ei[1], jnp.full((pad,), N, jnp.int32)])
    src = src.reshape(16, NCHT, CHUNK)
    dst = dst.reshape(16, NCHT, CHUNK)

    ones_rows = jnp.ones((CHUNK, D), jnp.float32)
    zrows = jnp.zeros((NPAD, D), jnp.float32)

    sc = 1.0 / jnp.sqrt(1.0 + EPS)
    st1 = jnp.stack([g1 * sc, b1 * g1 * sc + be1]).reshape(2, 1, D)
    st2 = jnp.stack([g2 * sc, b2 * g2 * sc + be2]).reshape(2, 1, D)

    deg_parts = _deg_kernel(dst, ones_rows, zrows)

    dis, h1 = pl.pallas_call(
        _tc1_body,
        grid=(GRID,),
        in_specs=[_row_spec(True),
                  _row_spec(), _full_w],
        out_specs=[_row_spec(), _row_spec()],
        out_shape=[jax.ShapeDtypeStruct((N, D), jnp.float32),
                   jax.ShapeDtypeStruct((N, D), jnp.float32)],
    )(deg_parts, x, W1)

    agg1 = _agg_kernel(h1, src, dst, zrows)

    st_spec = pl.BlockSpec((2, 1, D), lambda i: (0, 0, 0))
    x1, h2 = pl.pallas_call(
        functools.partial(_tc_mid_body, residual=False),
        grid=(GRID,),
        in_specs=[_row_spec(True), _row_spec(), _row_spec(), _full_w,
                  st_spec, _row_spec()],
        out_specs=[_row_spec(), _row_spec()],
        out_shape=[jax.ShapeDtypeStruct((N, D), jnp.float32),
                   jax.ShapeDtypeStruct((N, D), jnp.float32)],
    )(agg1, h1, dis, W2, st1, h1)

    agg2 = _agg_kernel(h2, src, dst, zrows)

    x2, h3 = pl.pallas_call(
        functools.partial(_tc_mid_body, residual=True),
        grid=(GRID,),
        in_specs=[_row_spec(True), _row_spec(), _row_spec(), _full_w,
                  st_spec, _row_spec()],
        out_specs=[_row_spec(), _row_spec()],
        out_shape=[jax.ShapeDtypeStruct((N, D), jnp.float32),
                   jax.ShapeDtypeStruct((N, D), jnp.float32)],
    )(agg2, h2, dis, W3, st2, x1)

    agg3 = _agg_kernel(h3, src, dst, zrows)

    out = pl.pallas_call(
        _tc4_body,
        grid=(GRID,),
        in_specs=[_row_spec(True), _row_spec(), _row_spec(), _vec_spec],
        out_specs=_row_spec(),
        out_shape=jax.ShapeDtypeStruct((N, D), jnp.float32),
    )(agg3, h3, dis, b3.reshape(1, D))

    return out


# speedup vs baseline: 21.4507x; 2.3026x over previous
import functools

import jax
import jax.numpy as jnp
from jax import lax
from jax.experimental import pallas as pl
from jax.experimental.pallas import tpu as pltpu
from jax.experimental.pallas import tpu_sc as plsc

N = 10000
E = 320000
D = 128
EPS = 1e-5

NTILES = 32
CHUNK = 128
G = 8
NGT = 20
NCHT = NGT * G
EPAD = 16 * NCHT * CHUNK
NG0 = 10
NG1 = NGT - NG0
NPAD = 10240
RPT = NPAD // 16
RBLK = 1000
GRID = N // RBLK

_mesh = plsc.VectorSubcoreMesh(core_axis_name="c", subcore_axis_name="s")


@functools.partial(
    pl.kernel,
    out_type=jax.ShapeDtypeStruct((2, NPAD, D), jnp.float32),
    mesh=_mesh,
    scratch_types=[
        pltpu.VMEM((G, CHUNK), jnp.int32),
        pltpu.VMEM((CHUNK, D), jnp.float32),
        pltpu.VMEM_SHARED((NPAD, D), jnp.float32),
    ],
)
def _deg_kernel(dst_hbm, ones_hbm, zrows_hbm, out_hbm, dst_g, ones_v, acc):
    cid = lax.axis_index("c")
    sid = lax.axis_index("s")
    gbase = cid * (NGT // 2)
    sl = pl.ds(sid * RPT, RPT)
    pltpu.sync_copy(ones_hbm, ones_v)
    pltpu.sync_copy(zrows_hbm.at[sl], acc.at[sl])
    plsc.subcore_barrier()

    def group(g, carry):
        pltpu.sync_copy(dst_hbm.at[sid, pl.ds((gbase + g) * G, G)], dst_g)

        def body(jj, c2):
            pltpu.sync_copy(ones_v, acc.at[dst_g.at[jj]], add=True)
            return c2

        return lax.fori_loop(0, G, body, carry)

    lax.fori_loop(0, NGT // 2, group, 0)
    plsc.subcore_barrier()
    pltpu.sync_copy(acc.at[sl], out_hbm.at[cid, sl])


@functools.partial(
    pl.kernel,
    out_type=jax.ShapeDtypeStruct((2, NPAD, D), jnp.float32),
    mesh=_mesh,
    scratch_types=[
        pltpu.VMEM((2, G, CHUNK), jnp.int32),
        pltpu.VMEM((2, G, CHUNK), jnp.int32),
        pltpu.VMEM((CHUNK, D), jnp.float32),
        pltpu.VMEM((CHUNK, D), jnp.float32),
        pltpu.VMEM_SHARED((NPAD, D), jnp.float32),
        pltpu.SemaphoreType.DMA,
        pltpu.SemaphoreType.DMA,
        pltpu.SemaphoreType.DMA,
        pltpu.SemaphoreType.DMA,
    ],
)
def _agg_kernel(h_hbm, src_hbm, dst_hbm, zrows_hbm, out_hbm,
                src_i, dst_i, rows0, rows1, acc, sg0, sg1, ss0, ss1):
    cid = lax.axis_index("c")
    sid = lax.axis_index("s")
    gbase = jnp.where(cid == 0, 0, NG0)
    ng = jnp.where(cid == 0, NG0, NG1)
    sl = pl.ds(sid * RPT, RPT)
    rows = (rows0, rows1)
    sgs = (sg0, sg1)
    sss = (ss0, ss1)

    pltpu.sync_copy(zrows_hbm.at[sl], acc.at[sl])
    pltpu.sync_copy(src_hbm.at[sid, pl.ds(gbase * G, G)], src_i.at[0])
    pltpu.sync_copy(dst_hbm.at[sid, pl.ds(gbase * G, G)], dst_i.at[0])
    plsc.subcore_barrier()
    pltpu.async_copy(h_hbm.at[src_i.at[0, 0]], rows0, sg0)

    def grp(g, carry):
        g2 = lax.rem(g, 2)
        ng2 = lax.rem(g + 1, 2)

        @pl.when(g + 1 < ng)
        def _():
            pltpu.sync_copy(src_hbm.at[sid, pl.ds((gbase + g + 1) * G, G)],
                            src_i.at[ng2])
            pltpu.sync_copy(dst_hbm.at[sid, pl.ds((gbase + g + 1) * G, G)],
                            dst_i.at[ng2])

        for jj in range(G):
            b = jj % 2
            nb = (jj + 1) % 2
            pltpu.make_async_copy(h_hbm.at[src_i.at[g2, jj]],
                                  rows[b], sgs[b]).wait()

            def wait_prev():
                pltpu.make_async_copy(
                    rows[nb], acc.at[dst_i.at[g2, jj]], sss[nb]).wait()

            if jj == 0:
                pl.when(g > 0)(wait_prev)
            else:
                wait_prev()

            if jj + 1 < G:
                pltpu.async_copy(h_hbm.at[src_i.at[g2, jj + 1]],
                                 rows[nb], sgs[nb])
            else:
                @pl.when(g + 1 < ng)
                def _():
                    pltpu.async_copy(h_hbm.at[src_i.at[ng2, 0]],
                                     rows[nb], sgs[nb])

            pltpu.async_copy(rows[b], acc.at[dst_i.at[g2, jj]],
                             sss[b], add=True)
        return carry

    lax.fori_loop(0, ng, grp, 0)
    pltpu.make_async_copy(rows1, acc.at[dst_i.at[0, 0]], ss1).wait()
    plsc.subcore_barrier()
    pltpu.sync_copy(acc.at[sl], out_hbm.at[cid, sl])


def _tc1_body(parts_ref, x_ref, w_ref, dis_ref, h_ref):
    p0 = parts_ref[0]
    p1 = parts_ref[1]
    deg = (p0 + p1)[:, 0:1] + 1.0
    dis = jnp.broadcast_to(lax.rsqrt(deg), (RBLK, D))
    h = jnp.dot(x_ref[...], w_ref[...], preferred_element_type=jnp.float32,
                precision=lax.Precision.HIGHEST)
    dis_ref[...] = dis
    h_ref[...] = dis * h


def _tc_mid_body(agg_ref, hp_ref, dis_ref, w_ref, st_ref, res_ref,
                 x_ref, h_ref, *, residual):
    a = agg_ref[0] + agg_ref[1] + hp_ref[...]
    dis = dis_ref[...]
    y = a * dis * st_ref[0] + st_ref[1]
    x = jnp.maximum(y, 0.0)
    if residual:
        x = x + res_ref[...]
    h = jnp.dot(x, w_ref[...], preferred_element_type=jnp.float32,
                precision=lax.Precision.HIGHEST)
    x_ref[...] = x
    h_ref[...] = dis * h


def _tc4_body(agg_ref, hp_ref, dis_ref, b_ref, out_ref):
    a = agg_ref[0] + agg_ref[1] + hp_ref[...]
    out_ref[...] = a * dis_ref[...] + b_ref[0]


def _row_spec(shape3=False):
    if shape3:
        return pl.BlockSpec((2, RBLK, D), lambda i: (0, i, 0))
    return pl.BlockSpec((RBLK, D), lambda i: (i, 0))


_full_w = pl.BlockSpec((D, D), lambda i: (0, 0))
_vec_spec = pl.BlockSpec((1, D), lambda i: (0, 0))


def kernel(x, edge_index, W1, b1, g1, be1, W2, b2, g2, be2, W3, b3):
    ei = edge_index.astype(jnp.int32)
    pad = EPAD - E
    pad_ar = jnp.arange(pad, dtype=jnp.int32)
    src = jnp.concatenate([ei[0], pad_ar % N])
    dst = jnp.concatenate([ei[1], N + pad_ar % (NPAD - N)])
    src = src.reshape(16, NCHT, CHUNK)
    dst = dst.reshape(16, NCHT, CHUNK)

    ones_rows = jnp.ones((CHUNK, D), jnp.float32)
    zrows = jnp.zeros((NPAD, D), jnp.float32)

    sc = 1.0 / jnp.sqrt(1.0 + EPS)
    st1 = jnp.stack([g1 * sc, b1 * g1 * sc + be1]).reshape(2, 1, D)
    st2 = jnp.stack([g2 * sc, b2 * g2 * sc + be2]).reshape(2, 1, D)

    deg_parts = _deg_kernel(dst, ones_rows, zrows)

    dis, h1 = pl.pallas_call(
        _tc1_body,
        grid=(GRID,),
        in_specs=[_row_spec(True),
                  _row_spec(), _full_w],
        out_specs=[_row_spec(), _row_spec()],
        out_shape=[jax.ShapeDtypeStruct((N, D), jnp.float32),
                   jax.ShapeDtypeStruct((N, D), jnp.float32)],
    )(deg_parts, x, W1)

    agg1 = _agg_kernel(h1, src, dst, zrows)

    st_spec = pl.BlockSpec((2, 1, D), lambda i: (0, 0, 0))
    x1, h2 = pl.pallas_call(
        functools.partial(_tc_mid_body, residual=False),
        grid=(GRID,),
        in_specs=[_row_spec(True), _row_spec(), _row_spec(), _full_w,
                  st_spec, _row_spec()],
        out_specs=[_row_spec(), _row_spec()],
        out_shape=[jax.ShapeDtypeStruct((N, D), jnp.float32),
                   jax.ShapeDtypeStruct((N, D), jnp.float32)],
    )(agg1, h1, dis, W2, st1, h1)

    agg2 = _agg_kernel(h2, src, dst, zrows)

    x2, h3 = pl.pallas_call(
        functools.partial(_tc_mid_body, residual=True),
        grid=(GRID,),
        in_specs=[_row_spec(True), _row_spec(), _row_spec(), _full_w,
                  st_spec, _row_spec()],
        out_specs=[_row_spec(), _row_spec()],
        out_shape=[jax.ShapeDtypeStruct((N, D), jnp.float32),
                   jax.ShapeDtypeStruct((N, D), jnp.float32)],
    )(agg2, h2, dis, W3, st2, x1)

    agg3 = _agg_kernel(h3, src, dst, zrows)

    out = pl.pallas_call(
        _tc4_body,
        grid=(GRID,),
        in_specs=[_row_spec(True), _row_spec(), _row_spec(), _vec_spec],
        out_specs=_row_spec(),
        out_shape=jax.ShapeDtypeStruct((N, D), jnp.float32),
    )(agg3, h3, dis, b3.reshape(1, D))

    return out


# speedup vs baseline: 21.8273x; 1.0176x over previous
import functools

import jax
import jax.numpy as jnp
from jax import lax
from jax.experimental import pallas as pl
from jax.experimental.pallas import tpu as pltpu
from jax.experimental.pallas import tpu_sc as plsc

N = 10000
E = 320000
D = 128
EPS = 1e-5

NTILES = 32
CHUNK = 128
G = 8
NGT = 20
NCHT = NGT * G
EPAD = 16 * NCHT * CHUNK
NG0 = 10
NG1 = NGT - NG0
NPAD = 10240
RPT = NPAD // 16
RBLK = 1000
GRID = N // RBLK

_mesh = plsc.VectorSubcoreMesh(core_axis_name="c", subcore_axis_name="s")


@functools.partial(
    pl.kernel,
    out_type=jax.ShapeDtypeStruct((2, NPAD, D), jnp.float32),
    mesh=_mesh,
    scratch_types=[
        pltpu.VMEM((2, G, CHUNK), jnp.int32),
        pltpu.VMEM((CHUNK, D), jnp.float32),
        pltpu.VMEM_SHARED((NPAD, D), jnp.float32),
        pltpu.SemaphoreType.DMA,
        pltpu.SemaphoreType.DMA,
    ],
)
def _deg_kernel(dst_hbm, ones_hbm, zrows_hbm, out_hbm, dst_i, ones_v, acc,
                sd0, sd1):
    cid = lax.axis_index("c")
    sid = lax.axis_index("s")
    gbase = cid * (NGT // 2)
    ngd = NGT // 2
    sl = pl.ds(sid * RPT, RPT)
    sds = (sd0, sd1)
    pltpu.sync_copy(ones_hbm, ones_v)
    pltpu.sync_copy(zrows_hbm.at[sl], acc.at[sl])
    plsc.subcore_barrier()

    def group(g, carry):
        g2 = lax.rem(g, 2)

        @pl.when(g > 1)
        def _():
            for jj in range(G):
                pltpu.make_async_copy(
                    ones_v, acc.at[dst_i.at[g2, jj]], sds[0]).wait()

        pltpu.sync_copy(dst_hbm.at[sid, pl.ds((gbase + g) * G, G)],
                        dst_i.at[g2])
        for jj in range(G):
            pltpu.async_copy(ones_v, acc.at[dst_i.at[g2, jj]], sds[0],
                             add=True)
        return carry

    lax.fori_loop(0, ngd, group, 0)
    for jj in range(2 * G):
        pltpu.make_async_copy(ones_v, acc.at[dst_i.at[0, 0]], sds[0]).wait()
    plsc.subcore_barrier()
    pltpu.sync_copy(acc.at[sl], out_hbm.at[cid, sl])


@functools.partial(
    pl.kernel,
    out_type=jax.ShapeDtypeStruct((2, NPAD, D), jnp.float32),
    mesh=_mesh,
    scratch_types=[
        pltpu.VMEM((2, G, CHUNK), jnp.int32),
        pltpu.VMEM((2, G, CHUNK), jnp.int32),
        pltpu.VMEM((CHUNK, D), jnp.float32),
        pltpu.VMEM((CHUNK, D), jnp.float32),
        pltpu.VMEM_SHARED((NPAD, D), jnp.float32),
        pltpu.SemaphoreType.DMA,
        pltpu.SemaphoreType.DMA,
        pltpu.SemaphoreType.DMA,
        pltpu.SemaphoreType.DMA,
    ],
)
def _agg_kernel(h_hbm, src_hbm, dst_hbm, zrows_hbm, out_hbm,
                src_i, dst_i, rows0, rows1, acc, sg0, sg1, ss0, ss1):
    cid = lax.axis_index("c")
    sid = lax.axis_index("s")
    gbase = jnp.where(cid == 0, 0, NG0)
    ng = jnp.where(cid == 0, NG0, NG1)
    sl = pl.ds(sid * RPT, RPT)
    rows = (rows0, rows1)
    sgs = (sg0, sg1)
    sss = (ss0, ss1)

    pltpu.sync_copy(zrows_hbm.at[sl], acc.at[sl])
    pltpu.sync_copy(src_hbm.at[sid, pl.ds(gbase * G, G)], src_i.at[0])
    pltpu.sync_copy(dst_hbm.at[sid, pl.ds(gbase * G, G)], dst_i.at[0])
    plsc.subcore_barrier()
    pltpu.async_copy(h_hbm.at[src_i.at[0, 0]], rows0, sg0)

    def grp(g, carry):
        g2 = lax.rem(g, 2)
        ng2 = lax.rem(g + 1, 2)

        @pl.when(g + 1 < ng)
        def _():
            pltpu.sync_copy(src_hbm.at[sid, pl.ds((gbase + g + 1) * G, G)],
                            src_i.at[ng2])
            pltpu.sync_copy(dst_hbm.at[sid, pl.ds((gbase + g + 1) * G, G)],
                            dst_i.at[ng2])

        for jj in range(G):
            b = jj % 2
            nb = (jj + 1) % 2
            pltpu.make_async_copy(h_hbm.at[src_i.at[g2, jj]],
                                  rows[b], sgs[b]).wait()

            def wait_prev():
                pltpu.make_async_copy(
                    rows[nb], acc.at[dst_i.at[g2, jj]], sss[nb]).wait()

            if jj == 0:
                pl.when(g > 0)(wait_prev)
            else:
                wait_prev()

            if jj + 1 < G:
                pltpu.async_copy(h_hbm.at[src_i.at[g2, jj + 1]],
                                 rows[nb], sgs[nb])
            else:
                @pl.when(g + 1 < ng)
                def _():
                    pltpu.async_copy(h_hbm.at[src_i.at[ng2, 0]],
                                     rows[nb], sgs[nb])

            pltpu.async_copy(rows[b], acc.at[dst_i.at[g2, jj]],
                             sss[b], add=True)
        return carry

    lax.fori_loop(0, ng, grp, 0)
    pltpu.make_async_copy(rows1, acc.at[dst_i.at[0, 0]], ss1).wait()
    plsc.subcore_barrier()
    pltpu.sync_copy(acc.at[sl], out_hbm.at[cid, sl])


def _tc_mm_body(x_ref, w_ref, h_ref):
    h_ref[...] = jnp.dot(x_ref[...], w_ref[...],
                         preferred_element_type=jnp.float32,
                         precision=lax.Precision.HIGHEST)


def _tc1_body(parts_ref, hraw_ref, dis_ref, h_ref):
    p0 = parts_ref[0]
    p1 = parts_ref[1]
    deg = (p0 + p1)[:, 0:1] + 1.0
    dis = jnp.broadcast_to(lax.rsqrt(deg), (RBLK, D))
    dis_ref[...] = dis
    h_ref[...] = dis * hraw_ref[...]


def _tc_mid_body(agg_ref, hp_ref, dis_ref, w_ref, st_ref, res_ref,
                 x_ref, h_ref, *, residual):
    a = agg_ref[0] + agg_ref[1] + hp_ref[...]
    dis = dis_ref[...]
    y = a * dis * st_ref[0] + st_ref[1]
    x = jnp.maximum(y, 0.0)
    if residual:
        x = x + res_ref[...]
    h = jnp.dot(x, w_ref[...], preferred_element_type=jnp.float32,
                precision=lax.Precision.HIGHEST)
    x_ref[...] = x
    h_ref[...] = dis * h


def _tc4_body(agg_ref, hp_ref, dis_ref, b_ref, out_ref):
    a = agg_ref[0] + agg_ref[1] + hp_ref[...]
    out_ref[...] = a * dis_ref[...] + b_ref[0]


def _row_spec(shape3=False):
    if shape3:
        return pl.BlockSpec((2, RBLK, D), lambda i: (0, i, 0))
    return pl.BlockSpec((RBLK, D), lambda i: (i, 0))


_full_w = pl.BlockSpec((D, D), lambda i: (0, 0))
_vec_spec = pl.BlockSpec((1, D), lambda i: (0, 0))


def kernel(x, edge_index, W1, b1, g1, be1, W2, b2, g2, be2, W3, b3):
    ei = edge_index.astype(jnp.int32)
    pad = EPAD - E
    pad_ar = jnp.arange(pad, dtype=jnp.int32)
    src = jnp.concatenate([ei[0], pad_ar % N])
    dst = jnp.concatenate([ei[1], N + pad_ar % (NPAD - N)])
    src = src.reshape(16, NCHT, CHUNK)
    dst = dst.reshape(16, NCHT, CHUNK)

    ones_rows = jnp.ones((CHUNK, D), jnp.float32)
    zrows = jnp.zeros((NPAD, D), jnp.float32)

    sc = 1.0 / jnp.sqrt(1.0 + EPS)
    st1 = jnp.stack([g1 * sc, b1 * g1 * sc + be1]).reshape(2, 1, D)
    st2 = jnp.stack([g2 * sc, b2 * g2 * sc + be2]).reshape(2, 1, D)

    deg_parts = _deg_kernel(dst, ones_rows, zrows)

    h1raw = pl.pallas_call(
        _tc_mm_body,
        grid=(GRID,),
        in_specs=[_row_spec(), _full_w],
        out_specs=_row_spec(),
        out_shape=jax.ShapeDtypeStruct((N, D), jnp.float32),
    )(x, W1)

    dis, h1 = pl.pallas_call(
        _tc1_body,
        grid=(GRID,),
        in_specs=[_row_spec(True), _row_spec()],
        out_specs=[_row_spec(), _row_spec()],
        out_shape=[jax.ShapeDtypeStruct((N, D), jnp.float32),
                   jax.ShapeDtypeStruct((N, D), jnp.float32)],
    )(deg_parts, h1raw)

    agg1 = _agg_kernel(h1, src, dst, zrows)

    st_spec = pl.BlockSpec((2, 1, D), lambda i: (0, 0, 0))
    x1, h2 = pl.pallas_call(
        functools.partial(_tc_mid_body, residual=False),
        grid=(GRID,),
        in_specs=[_row_spec(True), _row_spec(), _row_spec(), _full_w,
                  st_spec, _row_spec()],
        out_specs=[_row_spec(), _row_spec()],
        out_shape=[jax.ShapeDtypeStruct((N, D), jnp.float32),
                   jax.ShapeDtypeStruct((N, D), jnp.float32)],
    )(agg1, h1, dis, W2, st1, h1)

    agg2 = _agg_kernel(h2, src, dst, zrows)

    x2, h3 = pl.pallas_call(
        functools.partial(_tc_mid_body, residual=True),
        grid=(GRID,),
        in_specs=[_row_spec(True), _row_spec(), _row_spec(), _full_w,
                  st_spec, _row_spec()],
        out_specs=[_row_spec(), _row_spec()],
        out_shape=[jax.ShapeDtypeStruct((N, D), jnp.float32),
                   jax.ShapeDtypeStruct((N, D), jnp.float32)],
    )(agg2, h2, dis, W3, st2, x1)

    agg3 = _agg_kernel(h3, src, dst, zrows)

    out = pl.pallas_call(
        _tc4_body,
        grid=(GRID,),
        in_specs=[_row_spec(True), _row_spec(), _row_spec(), _vec_spec],
        out_specs=_row_spec(),
        out_shape=jax.ShapeDtypeStruct((N, D), jnp.float32),
    )(agg3, h3, dis, b3.reshape(1, D))

    return out


# speedup vs baseline: 21.8879x; 1.0028x over previous
import functools

import jax
import jax.numpy as jnp
from jax import lax
from jax.experimental import pallas as pl
from jax.experimental.pallas import tpu as pltpu
from jax.experimental.pallas import tpu_sc as plsc

N = 10000
E = 320000
D = 128
EPS = 1e-5

NTILES = 32
CHUNK = 128
G = 8
NGT = 20
NCHT = NGT * G
EPAD = 16 * NCHT * CHUNK
ACH = 64
ANGT = 40
ANCHT = ANGT * G
NG0 = 20
NG1 = ANGT - NG0
NPAD = 10240
RPT = NPAD // 16
RBLK = 1000
GRID = N // RBLK

_mesh = plsc.VectorSubcoreMesh(core_axis_name="c", subcore_axis_name="s")


@functools.partial(
    pl.kernel,
    out_type=jax.ShapeDtypeStruct((2, NPAD, D), jnp.float32),
    mesh=_mesh,
    scratch_types=[
        pltpu.VMEM((2, G, CHUNK), jnp.int32),
        pltpu.VMEM((CHUNK, D), jnp.float32),
        pltpu.VMEM_SHARED((NPAD, D), jnp.float32),
        pltpu.SemaphoreType.DMA,
        pltpu.SemaphoreType.DMA,
    ],
)
def _deg_kernel(dst_hbm, ones_hbm, zrows_hbm, out_hbm, dst_i, ones_v, acc,
                sd0, sd1):
    cid = lax.axis_index("c")
    sid = lax.axis_index("s")
    gbase = cid * (NGT // 2)
    ngd = NGT // 2
    sl = pl.ds(sid * RPT, RPT)
    sds = (sd0, sd1)
    pltpu.sync_copy(ones_hbm, ones_v)
    pltpu.sync_copy(zrows_hbm.at[sl], acc.at[sl])
    plsc.subcore_barrier()

    def group(g, carry):
        g2 = lax.rem(g, 2)

        @pl.when(g > 1)
        def _():
            for jj in range(G):
                pltpu.make_async_copy(
                    ones_v, acc.at[dst_i.at[g2, jj]], sds[0]).wait()

        pltpu.sync_copy(dst_hbm.at[sid, pl.ds((gbase + g) * G, G)],
                        dst_i.at[g2])
        for jj in range(G):
            pltpu.async_copy(ones_v, acc.at[dst_i.at[g2, jj]], sds[0],
                             add=True)
        return carry

    lax.fori_loop(0, ngd, group, 0)
    for jj in range(2 * G):
        pltpu.make_async_copy(ones_v, acc.at[dst_i.at[0, 0]], sds[0]).wait()
    plsc.subcore_barrier()
    pltpu.sync_copy(acc.at[sl], out_hbm.at[cid, sl])


@functools.partial(
    pl.kernel,
    out_type=jax.ShapeDtypeStruct((2, NPAD, D), jnp.float32),
    mesh=_mesh,
    scratch_types=[
        pltpu.VMEM((3, G, ACH), jnp.int32),
        pltpu.VMEM((3, G, ACH), jnp.int32),
        pltpu.VMEM((ACH, D), jnp.float32),
        pltpu.VMEM((ACH, D), jnp.float32),
        pltpu.VMEM((ACH, D), jnp.float32),
        pltpu.VMEM((ACH, D), jnp.float32),
        pltpu.VMEM_SHARED((NPAD, D), jnp.float32),
        pltpu.SemaphoreType.DMA,
        pltpu.SemaphoreType.DMA,
        pltpu.SemaphoreType.DMA,
        pltpu.SemaphoreType.DMA,
        pltpu.SemaphoreType.DMA,
        pltpu.SemaphoreType.DMA,
        pltpu.SemaphoreType.DMA,
        pltpu.SemaphoreType.DMA,
    ],
)
def _agg_kernel(h_hbm, src_hbm, dst_hbm, zrows_hbm, out_hbm,
                src_i, dst_i, r0, r1, r2, r3, acc,
                sg0, sg1, sg2, sg3, ss0, ss1, ss2, ss3):
    cid = lax.axis_index("c")
    sid = lax.axis_index("s")
    gbase = jnp.where(cid == 0, 0, NG0)
    ng = jnp.where(cid == 0, NG0, NG1)
    sl = pl.ds(sid * RPT, RPT)
    rows = (r0, r1, r2, r3)
    sgs = (sg0, sg1, sg2, sg3)
    sss = (ss0, ss1, ss2, ss3)

    pltpu.sync_copy(zrows_hbm.at[sl], acc.at[sl])
    pltpu.sync_copy(src_hbm.at[sid, pl.ds(gbase * G, G)], src_i.at[0])
    pltpu.sync_copy(dst_hbm.at[sid, pl.ds(gbase * G, G)], dst_i.at[0])
    plsc.subcore_barrier()
    pltpu.async_copy(h_hbm.at[src_i.at[0, 0]], rows[0], sgs[0])
    pltpu.async_copy(h_hbm.at[src_i.at[0, 1]], rows[1], sgs[1])

    def grp(g, carry):
        g3 = lax.rem(g, 3)
        ng3 = lax.rem(g + 1, 3)

        @pl.when(g + 1 < ng)
        def _():
            pltpu.sync_copy(src_hbm.at[sid, pl.ds((gbase + g + 1) * G, G)],
                            src_i.at[ng3])
            pltpu.sync_copy(dst_hbm.at[sid, pl.ds((gbase + g + 1) * G, G)],
                            dst_i.at[ng3])

        for jj in range(G):
            b = jj % 4
            fb = (jj + 2) % 4
            pltpu.make_async_copy(h_hbm.at[src_i.at[g3, jj]],
                                  rows[b], sgs[b]).wait()

            def wait_prev():
                pltpu.make_async_copy(
                    rows[fb], acc.at[dst_i.at[g3, jj]], sss[fb]).wait()

            if jj < 2:
                pl.when(g > 0)(wait_prev)
            else:
                wait_prev()

            if jj + 2 < G:
                pltpu.async_copy(h_hbm.at[src_i.at[g3, jj + 2]],
                                 rows[fb], sgs[fb])
            else:
                @pl.when(g + 1 < ng)
                def _(jj=jj, fb=fb):
                    pltpu.async_copy(h_hbm.at[src_i.at[ng3, jj + 2 - G]],
                                     rows[fb], sgs[fb])

            pltpu.async_copy(rows[b], acc.at[dst_i.at[g3, jj]],
                             sss[b], add=True)
        return carry

    lax.fori_loop(0, ng, grp, 0)
    pltpu.make_async_copy(rows[2], acc.at[dst_i.at[0, 0]], sss[2]).wait()
    pltpu.make_async_copy(rows[3], acc.at[dst_i.at[0, 0]], sss[3]).wait()
    plsc.subcore_barrier()
    pltpu.sync_copy(acc.at[sl], out_hbm.at[cid, sl])


def _tc_mm_body(x_ref, w_ref, h_ref):
    h_ref[...] = jnp.dot(x_ref[...], w_ref[...],
                         preferred_element_type=jnp.float32,
                         precision=lax.Precision.HIGHEST)


def _tc1_body(parts_ref, hraw_ref, dis_ref, h_ref):
    p0 = parts_ref[0]
    p1 = parts_ref[1]
    deg = (p0 + p1)[:, 0:1] + 1.0
    dis = jnp.broadcast_to(lax.rsqrt(deg), (RBLK, D))
    dis_ref[...] = dis
    h_ref[...] = dis * hraw_ref[...]


def _tc_mid_body(agg_ref, hp_ref, dis_ref, w_ref, st_ref, res_ref,
                 x_ref, h_ref, *, residual):
    a = agg_ref[0] + agg_ref[1] + hp_ref[...]
    dis = dis_ref[...]
    y = a * dis * st_ref[0] + st_ref[1]
    x = jnp.maximum(y, 0.0)
    if residual:
        x = x + res_ref[...]
    h = jnp.dot(x, w_ref[...], preferred_element_type=jnp.float32,
                precision=lax.Precision.HIGHEST)
    x_ref[...] = x
    h_ref[...] = dis * h


def _tc4_body(agg_ref, hp_ref, dis_ref, b_ref, out_ref):
    a = agg_ref[0] + agg_ref[1] + hp_ref[...]
    out_ref[...] = a * dis_ref[...] + b_ref[0]


def _row_spec(shape3=False):
    if shape3:
        return pl.BlockSpec((2, RBLK, D), lambda i: (0, i, 0))
    return pl.BlockSpec((RBLK, D), lambda i: (i, 0))


_full_w = pl.BlockSpec((D, D), lambda i: (0, 0))
_vec_spec = pl.BlockSpec((1, D), lambda i: (0, 0))


def kernel(x, edge_index, W1, b1, g1, be1, W2, b2, g2, be2, W3, b3):
    ei = edge_index.astype(jnp.int32)
    pad = EPAD - E
    pad_ar = jnp.arange(pad, dtype=jnp.int32)
    src = jnp.concatenate([ei[0], pad_ar % N])
    dst = jnp.concatenate([ei[1], N + pad_ar % (NPAD - N)])
    src64 = src.reshape(16, ANCHT, ACH)
    dst64 = dst.reshape(16, ANCHT, ACH)
    dst128 = dst.reshape(16, NCHT, CHUNK)

    ones_rows = jnp.ones((CHUNK, D), jnp.float32)
    zrows = jnp.zeros((NPAD, D), jnp.float32)

    sc = 1.0 / jnp.sqrt(1.0 + EPS)
    st1 = jnp.stack([g1 * sc, b1 * g1 * sc + be1]).reshape(2, 1, D)
    st2 = jnp.stack([g2 * sc, b2 * g2 * sc + be2]).reshape(2, 1, D)

    deg_parts = _deg_kernel(dst128, ones_rows, zrows)

    h1raw = pl.pallas_call(
        _tc_mm_body,
        grid=(GRID,),
        in_specs=[_row_spec(), _full_w],
        out_specs=_row_spec(),
        out_shape=jax.ShapeDtypeStruct((N, D), jnp.float32),
    )(x, W1)

    dis, h1 = pl.pallas_call(
        _tc1_body,
        grid=(GRID,),
        in_specs=[_row_spec(True), _row_spec()],
        out_specs=[_row_spec(), _row_spec()],
        out_shape=[jax.ShapeDtypeStruct((N, D), jnp.float32),
                   jax.ShapeDtypeStruct((N, D), jnp.float32)],
    )(deg_parts, h1raw)

    agg1 = _agg_kernel(h1, src64, dst64, zrows)

    st_spec = pl.BlockSpec((2, 1, D), lambda i: (0, 0, 0))
    x1, h2 = pl.pallas_call(
        functools.partial(_tc_mid_body, residual=False),
        grid=(GRID,),
        in_specs=[_row_spec(True), _row_spec(), _row_spec(), _full_w,
                  st_spec, _row_spec()],
        out_specs=[_row_spec(), _row_spec()],
        out_shape=[jax.ShapeDtypeStruct((N, D), jnp.float32),
                   jax.ShapeDtypeStruct((N, D), jnp.float32)],
    )(agg1, h1, dis, W2, st1, h1)

    agg2 = _agg_kernel(h2, src64, dst64, zrows)

    x2, h3 = pl.pallas_call(
        functools.partial(_tc_mid_body, residual=True),
        grid=(GRID,),
        in_specs=[_row_spec(True), _row_spec(), _row_spec(), _full_w,
                  st_spec, _row_spec()],
        out_specs=[_row_spec(), _row_spec()],
        out_shape=[jax.ShapeDtypeStruct((N, D), jnp.float32),
                   jax.ShapeDtypeStruct((N, D), jnp.float32)],
    )(agg2, h2, dis, W3, st2, x1)

    agg3 = _agg_kernel(h3, src64, dst64, zrows)

    out = pl.pallas_call(
        _tc4_body,
        grid=(GRID,),
        in_specs=[_row_spec(True), _row_spec(), _row_spec(), _vec_spec],
        out_specs=_row_spec(),
        out_shape=jax.ShapeDtypeStruct((N, D), jnp.float32),
    )(agg3, h3, dis, b3.reshape(1, D))

    return out


# speedup vs baseline: 22.6193x; 1.0334x over previous
import functools

import jax
import jax.numpy as jnp
from jax import lax
from jax.experimental import pallas as pl
from jax.experimental.pallas import tpu as pltpu
from jax.experimental.pallas import tpu_sc as plsc

N = 10000
E = 320000
D = 128
EPS = 1e-5

NTILES = 32
CHUNK = 128
G = 8
NGT = 20
NCHT = NGT * G
EPAD = 16 * NCHT * CHUNK
ACH = 64
AG = 16
ANGT = 20
ANCHT = ANGT * AG
NG0 = 10
NG1 = ANGT - NG0
NPAD = 10240
RPT = NPAD // 16
RBLK = 1000
GRID = N // RBLK

_mesh = plsc.VectorSubcoreMesh(core_axis_name="c", subcore_axis_name="s")


@functools.partial(
    pl.kernel,
    out_type=jax.ShapeDtypeStruct((2, NPAD, D), jnp.float32),
    mesh=_mesh,
    scratch_types=[
        pltpu.VMEM((2, G, CHUNK), jnp.int32),
        pltpu.VMEM((CHUNK, D), jnp.float32),
        pltpu.VMEM_SHARED((NPAD, D), jnp.float32),
        pltpu.SemaphoreType.DMA,
        pltpu.SemaphoreType.DMA,
    ],
)
def _deg_kernel(dst_hbm, ones_hbm, zrows_hbm, out_hbm, dst_i, ones_v, acc,
                sd0, sd1):
    cid = lax.axis_index("c")
    sid = lax.axis_index("s")
    gbase = cid * (NGT // 2)
    ngd = NGT // 2
    sl = pl.ds(sid * RPT, RPT)
    sds = (sd0, sd1)
    pltpu.sync_copy(ones_hbm, ones_v)
    pltpu.sync_copy(zrows_hbm.at[sl], acc.at[sl])
    plsc.subcore_barrier()

    def group(g, carry):
        g2 = lax.rem(g, 2)

        @pl.when(g > 1)
        def _():
            for jj in range(G):
                pltpu.make_async_copy(
                    ones_v, acc.at[dst_i.at[g2, jj]], sds[0]).wait()

        pltpu.sync_copy(dst_hbm.at[sid, pl.ds((gbase + g) * G, G)],
                        dst_i.at[g2])
        for jj in range(G):
            pltpu.async_copy(ones_v, acc.at[dst_i.at[g2, jj]], sds[0],
                             add=True)
        return carry

    lax.fori_loop(0, ngd, group, 0)
    for jj in range(2 * G):
        pltpu.make_async_copy(ones_v, acc.at[dst_i.at[0, 0]], sds[0]).wait()
    plsc.subcore_barrier()
    pltpu.sync_copy(acc.at[sl], out_hbm.at[cid, sl])


@functools.partial(
    pl.kernel,
    out_type=jax.ShapeDtypeStruct((2, NPAD, D), jnp.float32),
    mesh=_mesh,
    scratch_types=[
        pltpu.VMEM((3, AG, ACH), jnp.int32),
        pltpu.VMEM((3, AG, ACH), jnp.int32),
        pltpu.VMEM((ACH, D), jnp.float32),
        pltpu.VMEM((ACH, D), jnp.float32),
        pltpu.VMEM((ACH, D), jnp.float32),
        pltpu.VMEM((ACH, D), jnp.float32),
        pltpu.VMEM_SHARED((NPAD, D), jnp.float32),
        pltpu.SemaphoreType.DMA,
        pltpu.SemaphoreType.DMA,
        pltpu.SemaphoreType.DMA,
        pltpu.SemaphoreType.DMA,
        pltpu.SemaphoreType.DMA,
        pltpu.SemaphoreType.DMA,
        pltpu.SemaphoreType.DMA,
        pltpu.SemaphoreType.DMA,
        pltpu.SemaphoreType.DMA,
    ],
)
def _agg_kernel(h_hbm, src_hbm, dst_hbm, zrows_hbm, out_hbm,
                src_i, dst_i, r0, r1, r2, r3, acc,
                sg0, sg1, sg2, sg3, ss0, ss1, ss2, ss3, si):
    cid = lax.axis_index("c")
    sid = lax.axis_index("s")
    gbase = jnp.where(cid == 0, 0, NG0)
    ng = jnp.where(cid == 0, NG0, NG1)
    sl = pl.ds(sid * RPT, RPT)
    rows = (r0, r1, r2, r3)
    sgs = (sg0, sg1, sg2, sg3)
    sss = (ss0, ss1, ss2, ss3)

    pltpu.sync_copy(zrows_hbm.at[sl], acc.at[sl])
    pltpu.sync_copy(src_hbm.at[sid, pl.ds(gbase * AG, AG)], src_i.at[0])
    pltpu.sync_copy(dst_hbm.at[sid, pl.ds(gbase * AG, AG)], dst_i.at[0])
    plsc.subcore_barrier()
    pltpu.async_copy(h_hbm.at[src_i.at[0, 0]], rows[0], sgs[0])
    pltpu.async_copy(h_hbm.at[src_i.at[0, 1]], rows[1], sgs[1])

    def grp(g, carry):
        g3 = lax.rem(g, 3)
        ng3 = lax.rem(g + 1, 3)

        @pl.when(g + 1 < ng)
        def _():
            pltpu.async_copy(src_hbm.at[sid, pl.ds((gbase + g + 1) * AG, AG)],
                             src_i.at[ng3], si)
            pltpu.async_copy(dst_hbm.at[sid, pl.ds((gbase + g + 1) * AG, AG)],
                             dst_i.at[ng3], si)

        for jj in range(AG):
            b = jj % 4
            fb = (jj + 2) % 4
            pltpu.make_async_copy(h_hbm.at[src_i.at[g3, jj]],
                                  rows[b], sgs[b]).wait()

            def wait_prev():
                pltpu.make_async_copy(
                    rows[fb], acc.at[dst_i.at[g3, jj]], sss[fb]).wait()

            if jj < 2:
                pl.when(g > 0)(wait_prev)
            else:
                wait_prev()

            if jj + 2 < AG:
                pltpu.async_copy(h_hbm.at[src_i.at[g3, jj + 2]],
                                 rows[fb], sgs[fb])
            else:
                if jj + 2 == AG:
                    @pl.when(g + 1 < ng)
                    def _():
                        pltpu.make_async_copy(
                            src_hbm.at[sid, pl.ds((gbase + g + 1) * AG, AG)],
                            src_i.at[ng3], si).wait()
                        pltpu.make_async_copy(
                            dst_hbm.at[sid, pl.ds((gbase + g + 1) * AG, AG)],
                            dst_i.at[ng3], si).wait()

                @pl.when(g + 1 < ng)
                def _(jj=jj, fb=fb):
                    pltpu.async_copy(h_hbm.at[src_i.at[ng3, jj + 2 - AG]],
                                     rows[fb], sgs[fb])

            pltpu.async_copy(rows[b], acc.at[dst_i.at[g3, jj]],
                             sss[b], add=True)
        return carry

    lax.fori_loop(0, ng, grp, 0)
    pltpu.make_async_copy(rows[2], acc.at[dst_i.at[0, 0]], sss[2]).wait()
    pltpu.make_async_copy(rows[3], acc.at[dst_i.at[0, 0]], sss[3]).wait()
    plsc.subcore_barrier()
    pltpu.sync_copy(acc.at[sl], out_hbm.at[cid, sl])


def _tc_mm_body(x_ref, w_ref, h_ref):
    h_ref[...] = jnp.dot(x_ref[...], w_ref[...],
                         preferred_element_type=jnp.float32,
                         precision=lax.Precision.HIGHEST)


def _tc1_body(parts_ref, hraw_ref, dis_ref, h_ref):
    p0 = parts_ref[0]
    p1 = parts_ref[1]
    deg = (p0 + p1)[:, 0:1] + 1.0
    dis = jnp.broadcast_to(lax.rsqrt(deg), (RBLK, D))
    dis_ref[...] = dis
    h_ref[...] = dis * hraw_ref[...]


def _tc_mid_body(agg_ref, hp_ref, dis_ref, w_ref, st_ref, res_ref,
                 x_ref, h_ref, *, residual):
    a = agg_ref[0] + agg_ref[1] + hp_ref[...]
    dis = dis_ref[...]
    y = a * dis * st_ref[0] + st_ref[1]
    x = jnp.maximum(y, 0.0)
    if residual:
        x = x + res_ref[...]
    h = jnp.dot(x, w_ref[...], preferred_element_type=jnp.float32,
                precision=lax.Precision.HIGHEST)
    x_ref[...] = x
    h_ref[...] = dis * h


def _tc4_body(agg_ref, hp_ref, dis_ref, b_ref, out_ref):
    a = agg_ref[0] + agg_ref[1] + hp_ref[...]
    out_ref[...] = a * dis_ref[...] + b_ref[0]


def _row_spec(shape3=False):
    if shape3:
        return pl.BlockSpec((2, RBLK, D), lambda i: (0, i, 0))
    return pl.BlockSpec((RBLK, D), lambda i: (i, 0))


_full_w = pl.BlockSpec((D, D), lambda i: (0, 0))
_vec_spec = pl.BlockSpec((1, D), lambda i: (0, 0))


def kernel(x, edge_index, W1, b1, g1, be1, W2, b2, g2, be2, W3, b3):
    ei = edge_index.astype(jnp.int32)
    pad = EPAD - E
    pad_ar = jnp.arange(pad, dtype=jnp.int32)
    src = jnp.concatenate([ei[0], pad_ar % N])
    dst = jnp.concatenate([ei[1], N + pad_ar % (NPAD - N)])
    src64 = src.reshape(16, ANCHT, ACH)
    dst64 = dst.reshape(16, ANCHT, ACH)
    dst128 = dst.reshape(16, NCHT, CHUNK)

    ones_rows = jnp.ones((CHUNK, D), jnp.float32)
    zrows = jnp.zeros((NPAD, D), jnp.float32)

    sc = 1.0 / jnp.sqrt(1.0 + EPS)
    st1 = jnp.stack([g1 * sc, b1 * g1 * sc + be1]).reshape(2, 1, D)
    st2 = jnp.stack([g2 * sc, b2 * g2 * sc + be2]).reshape(2, 1, D)

    deg_parts = _deg_kernel(dst128, ones_rows, zrows)

    h1raw = pl.pallas_call(
        _tc_mm_body,
        grid=(GRID,),
        in_specs=[_row_spec(), _full_w],
        out_specs=_row_spec(),
        out_shape=jax.ShapeDtypeStruct((N, D), jnp.float32),
    )(x, W1)

    dis, h1 = pl.pallas_call(
        _tc1_body,
        grid=(GRID,),
        in_specs=[_row_spec(True), _row_spec()],
        out_specs=[_row_spec(), _row_spec()],
        out_shape=[jax.ShapeDtypeStruct((N, D), jnp.float32),
                   jax.ShapeDtypeStruct((N, D), jnp.float32)],
    )(deg_parts, h1raw)

    agg1 = _agg_kernel(h1, src64, dst64, zrows)

    st_spec = pl.BlockSpec((2, 1, D), lambda i: (0, 0, 0))
    x1, h2 = pl.pallas_call(
        functools.partial(_tc_mid_body, residual=False),
        grid=(GRID,),
        in_specs=[_row_spec(True), _row_spec(), _row_spec(), _full_w,
                  st_spec, _row_spec()],
        out_specs=[_row_spec(), _row_spec()],
        out_shape=[jax.ShapeDtypeStruct((N, D), jnp.float32),
                   jax.ShapeDtypeStruct((N, D), jnp.float32)],
    )(agg1, h1, dis, W2, st1, h1)

    agg2 = _agg_kernel(h2, src64, dst64, zrows)

    x2, h3 = pl.pallas_call(
        functools.partial(_tc_mid_body, residual=True),
        grid=(GRID,),
        in_specs=[_row_spec(True), _row_spec(), _row_spec(), _full_w,
                  st_spec, _row_spec()],
        out_specs=[_row_spec(), _row_spec()],
        out_shape=[jax.ShapeDtypeStruct((N, D), jnp.float32),
                   jax.ShapeDtypeStruct((N, D), jnp.float32)],
    )(agg2, h2, dis, W3, st2, x1)

    agg3 = _agg_kernel(h3, src64, dst64, zrows)

    out = pl.pallas_call(
        _tc4_body,
        grid=(GRID,),
        in_specs=[_row_spec(True), _row_spec(), _row_spec(), _vec_spec],
        out_specs=_row_spec(),
        out_shape=jax.ShapeDtypeStruct((N, D), jnp.float32),
    )(agg3, h3, dis, b3.reshape(1, D))

    return out
